# baseline (speedup 1.0000x reference)
import jax
import jax.numpy as jnp
from jax.experimental import pallas as pl
from jax.experimental.pallas import tpu as pltpu

_N_ROWS = 128
_N_COLS = 100000
_BLOCK_ROWS = 8
_GRID = _N_ROWS // _BLOCK_ROWS

_CHUNK = 8192
_NFULL = _N_COLS // _CHUNK
_TAIL_START = _NFULL * _CHUNK
_TAIL = _N_COLS - _TAIL_START

_ROT_A = (13, 15, 26, 6)
_ROT_B = (17, 29, 16, 24)
_KS = (0, 1, 0x1BD11BDB)


def _rotl(x, r):
    return (x << jnp.uint32(r)) | (x >> jnp.uint32(32 - r))


def _threefry_bits(j):
    ks = tuple(jnp.uint32(k) for k in _KS)
    x1 = j + ks[1]
    x0 = x1
    x1 = _rotl(x1, _ROT_A[0]) ^ x0
    for r in _ROT_A[1:]:
        x0 = x0 + x1
        x1 = _rotl(x1, r)
        x1 = x0 ^ x1
    inject = ((ks[1], ks[2], 1), (ks[2], ks[0], 2), (ks[0], ks[1], 3),
              (ks[1], ks[2], 4), (ks[2], ks[0], 5))
    rots = (_ROT_B, _ROT_A, _ROT_B, _ROT_A)
    for (ka, kb, c), rgroup in zip(inject, rots + ((),)):
        x0 = x0 + ka
        x1 = x1 + kb + jnp.uint32(c)
        for r in rgroup:
            x0 = x0 + x1
            x1 = _rotl(x1, r)
            x1 = x0 ^ x1
    return x0 ^ x1


def _z_chunk(x, idx, rt):
    bits = _threefry_bits(idx)
    f = jax.lax.bitcast_convert_type(
        (bits >> jnp.uint32(9)) | jnp.uint32(0x3F800000), jnp.float32)
    u = f - jnp.float32(1.0)
    eps = jnp.float32(1e-20)
    g = -jnp.log(-jnp.log(u + eps) + eps)
    return (x + g) * rt


def _compute_block(x_ref, o_ref, i, rt):
    shape = (_BLOCK_ROWS, _CHUNK)
    row = jax.lax.broadcasted_iota(jnp.uint32, shape, 0)
    col = jax.lax.broadcasted_iota(jnp.uint32, shape, 1)
    base_row = (i * _BLOCK_ROWS).astype(jnp.uint32)
    idx0 = (base_row + row) * jnp.uint32(_N_COLS) + col

    def p1(k, m):
        off = k * _CHUNK
        z = _z_chunk(x_ref[:, pl.ds(off, _CHUNK)],
                     idx0 + off.astype(jnp.uint32), rt)
        o_ref[:, pl.ds(off, _CHUNK)] = z
        return jnp.maximum(m, z)

    m_lanes = jax.lax.fori_loop(
        0, _NFULL, p1, jnp.full(shape, -jnp.inf, jnp.float32))
    z_tail = _z_chunk(x_ref[:, _TAIL_START:],
                      idx0[:, :_TAIL] + jnp.uint32(_TAIL_START), rt)
    o_ref[:, _TAIL_START:] = z_tail
    m = jnp.maximum(jnp.max(m_lanes, axis=-1, keepdims=True),
                    jnp.max(z_tail, axis=-1, keepdims=True))

    def p2(k, s):
        off = k * _CHUNK
        e = jnp.exp(o_ref[:, pl.ds(off, _CHUNK)] - m)
        o_ref[:, pl.ds(off, _CHUNK)] = e
        return s + e

    s_lanes = jax.lax.fori_loop(0, _NFULL, p2, jnp.zeros(shape, jnp.float32))
    e_tail = jnp.exp(z_tail - m)
    o_ref[:, _TAIL_START:] = e_tail
    s = (jnp.sum(s_lanes, axis=-1, keepdims=True)
         + jnp.sum(e_tail, axis=-1, keepdims=True))
    rs = jnp.float32(1.0) / s

    def p3(k, carry):
        off = k * _CHUNK
        o_ref[:, pl.ds(off, _CHUNK)] = o_ref[:, pl.ds(off, _CHUNK)] * rs
        return carry

    jax.lax.fori_loop(0, _NFULL, p3, 0)
    o_ref[:, _TAIL_START:] = e_tail * rs


def _in_copy(x_hbm, x_buf, in_sems, block, slot):
    return pltpu.make_async_copy(
        x_hbm.at[pl.ds(block * _BLOCK_ROWS, _BLOCK_ROWS), :],
        x_buf.at[slot], in_sems.at[slot])


def _out_copy(o_hbm, o_buf, out_sems, block, slot):
    return pltpu.make_async_copy(
        o_buf.at[slot],
        o_hbm.at[pl.ds(block * _BLOCK_ROWS, _BLOCK_ROWS), :],
        out_sems.at[slot])


def _body(x_hbm, t_ref, o_hbm, x_buf, o_buf, in_sems, out_sems):
    i = pl.program_id(0)
    rt = jnp.float32(1.0) / t_ref[0].astype(jnp.float32)
    slot = jax.lax.rem(i, 2)
    nslot = jax.lax.rem(i + 1, 2)

    @pl.when(i == 0)
    def _():
        _in_copy(x_hbm, x_buf, in_sems, i, slot).start()

    @pl.when(i + 1 < _GRID)
    def _():
        _in_copy(x_hbm, x_buf, in_sems, i + 1, nslot).start()

    _in_copy(x_hbm, x_buf, in_sems, i, slot).wait()

    @pl.when(i >= 2)
    def _():
        _out_copy(o_hbm, o_buf, out_sems, i - 2, slot).wait()

    _compute_block(x_buf.at[slot], o_buf.at[slot], i, rt)

    _out_copy(o_hbm, o_buf, out_sems, i, slot).start()

    @pl.when(i == _GRID - 1)
    def _():
        _out_copy(o_hbm, o_buf, out_sems, i - 1, nslot).wait()
        _out_copy(o_hbm, o_buf, out_sems, i, slot).wait()


def kernel(logits, temperature, use_gpu):
    del use_gpu
    t = jnp.reshape(temperature, (1,))
    return pl.pallas_call(
        _body,
        grid=(_GRID,),
        in_specs=[
            pl.BlockSpec(memory_space=pl.ANY),
            pl.BlockSpec(memory_space=pltpu.SMEM),
        ],
        out_specs=pl.BlockSpec(memory_space=pl.ANY),
        out_shape=jax.ShapeDtypeStruct((_N_ROWS, _N_COLS), jnp.float32),
        scratch_shapes=[
            pltpu.VMEM((2, _BLOCK_ROWS, _N_COLS), jnp.float32),
            pltpu.VMEM((2, _BLOCK_ROWS, _N_COLS), jnp.float32),
            pltpu.SemaphoreType.DMA((2,)),
            pltpu.SemaphoreType.DMA((2,)),
        ],
    )(logits, t)


# speedup vs baseline: 1.0017x; 1.0017x over previous
import jax
import jax.numpy as jnp
from jax.experimental import pallas as pl
from jax.experimental.pallas import tpu as pltpu

_N_ROWS = 128
_N_COLS = 100000
_BLOCK_ROWS = 8
_GRID = _N_ROWS // _BLOCK_ROWS

_CHUNK = 8192
_NFULL = _N_COLS // _CHUNK
_TAIL_START = _NFULL * _CHUNK
_TAIL = _N_COLS - _TAIL_START

_ROT_A = (13, 15, 26, 6)
_ROT_B = (17, 29, 16, 24)
_KS = (0, 1, 0x1BD11BDB)


def _rotl(x, r):
    return (x << jnp.uint32(r)) | (x >> jnp.uint32(32 - r))


def _threefry_bits(j):
    ks = tuple(jnp.uint32(k) for k in _KS)
    x1 = j + ks[1]
    x0 = x1
    x1 = _rotl(x1, _ROT_A[0]) ^ x0
    for r in _ROT_A[1:]:
        x0 = x0 + x1
        x1 = _rotl(x1, r)
        x1 = x0 ^ x1
    inject = ((ks[1], ks[2], 1), (ks[2], ks[0], 2), (ks[0], ks[1], 3),
              (ks[1], ks[2], 4), (ks[2], ks[0], 5))
    rots = (_ROT_B, _ROT_A, _ROT_B, _ROT_A)
    for (ka, kb, c), rgroup in zip(inject, rots + ((),)):
        x0 = x0 + ka
        x1 = x1 + kb + jnp.uint32(c)
        for r in rgroup:
            x0 = x0 + x1
            x1 = _rotl(x1, r)
            x1 = x0 ^ x1
    return x0 ^ x1


def _z_chunk(x, idx, rt):
    bits = _threefry_bits(idx)
    f = jax.lax.bitcast_convert_type(
        (bits >> jnp.uint32(9)) | jnp.uint32(0x3F800000), jnp.float32)
    u = f - jnp.float32(1.0)
    eps = jnp.float32(1e-20)
    g = -jnp.log(-jnp.log(u + eps) + eps)
    return (x + g) * rt


def _compute_block(x_ref, o_ref, i, rt):
    shape = (_BLOCK_ROWS, _CHUNK)
    row = jax.lax.broadcasted_iota(jnp.uint32, shape, 0)
    col = jax.lax.broadcasted_iota(jnp.uint32, shape, 1)
    base_row = (i * _BLOCK_ROWS).astype(jnp.uint32)
    idx0 = (base_row + row) * jnp.uint32(_N_COLS) + col

    def p1(k, m):
        off = k * _CHUNK
        z = _z_chunk(x_ref[:, pl.ds(off, _CHUNK)],
                     idx0 + off.astype(jnp.uint32), rt)
        o_ref[:, pl.ds(off, _CHUNK)] = z
        return jnp.maximum(m, z)

    m_lanes = jax.lax.fori_loop(
        0, _NFULL, p1, jnp.full(shape, -jnp.inf, jnp.float32))
    z_tail = _z_chunk(x_ref[:, _TAIL_START:],
                      idx0[:, :_TAIL] + jnp.uint32(_TAIL_START), rt)
    o_ref[:, _TAIL_START:] = z_tail
    m = jnp.maximum(jnp.max(m_lanes, axis=-1, keepdims=True),
                    jnp.max(z_tail, axis=-1, keepdims=True))

    def p2(k, s):
        off = k * _CHUNK
        e = jnp.exp(o_ref[:, pl.ds(off, _CHUNK)] - m)
        o_ref[:, pl.ds(off, _CHUNK)] = e
        return s + e

    s_lanes = jax.lax.fori_loop(0, _NFULL, p2, jnp.zeros(shape, jnp.float32))
    e_tail = jnp.exp(z_tail - m)
    o_ref[:, _TAIL_START:] = e_tail
    s = (jnp.sum(s_lanes, axis=-1, keepdims=True)
         + jnp.sum(e_tail, axis=-1, keepdims=True))
    rs = jnp.float32(1.0) / s

    def p3(k, carry):
        off = k * _CHUNK
        o_ref[:, pl.ds(off, _CHUNK)] = o_ref[:, pl.ds(off, _CHUNK)] * rs
        return carry

    jax.lax.fori_loop(0, _NFULL, p3, 0)
    o_ref[:, _TAIL_START:] = e_tail * rs


def _in_copy(x_hbm, x_buf, in_sems, block, slot):
    return pltpu.make_async_copy(
        x_hbm.at[pl.ds(block * _BLOCK_ROWS, _BLOCK_ROWS), :],
        x_buf.at[slot], in_sems.at[slot])


def _out_copy(o_hbm, o_buf, out_sems, block, slot):
    return pltpu.make_async_copy(
        o_buf.at[slot],
        o_hbm.at[pl.ds(block * _BLOCK_ROWS, _BLOCK_ROWS), :],
        out_sems.at[slot])


def _body(x_hbm, t_ref, o_hbm, x_buf, o_buf, in_sems, out_sems):
    i = pl.program_id(0)
    rt = jnp.float32(1.0) / t_ref[0].astype(jnp.float32)
    slot = jax.lax.rem(i, 2)
    nslot = jax.lax.rem(i + 1, 2)

    @pl.when(i == 0)
    def _():
        _in_copy(x_hbm, x_buf, in_sems, i, slot).start()

    @pl.when(i + 1 < _GRID)
    def _():
        _in_copy(x_hbm, x_buf, in_sems, i + 1, nslot).start()

    _in_copy(x_hbm, x_buf, in_sems, i, slot).wait()

    @pl.when(i >= 2)
    def _():
        _out_copy(o_hbm, o_buf, out_sems, i - 2, slot).wait()

    _compute_block(x_buf.at[slot], o_buf.at[slot], i, rt)

    _out_copy(o_hbm, o_buf, out_sems, i, slot).start()

    @pl.when(i == _GRID - 1)
    def _():
        _out_copy(o_hbm, o_buf, out_sems, i - 1, nslot).wait()
        _out_copy(o_hbm, o_buf, out_sems, i, slot).wait()


def kernel(logits, temperature, use_gpu):
    del use_gpu
    t = jnp.reshape(temperature, (1,))
    return pl.pallas_call(
        _body,
        grid=(_GRID,),
        in_specs=[
            pl.BlockSpec(memory_space=pl.ANY),
            pl.BlockSpec(memory_space=pltpu.SMEM),
        ],
        out_specs=pl.BlockSpec(memory_space=pl.ANY),
        out_shape=jax.ShapeDtypeStruct((_N_ROWS, _N_COLS), jnp.float32),
        compiler_params=pltpu.CompilerParams(needs_layout_passes=False),
        scratch_shapes=[
            pltpu.VMEM((2, _BLOCK_ROWS, _N_COLS), jnp.float32),
            pltpu.VMEM((2, _BLOCK_ROWS, _N_COLS), jnp.float32),
            pltpu.SemaphoreType.DMA((2,)),
            pltpu.SemaphoreType.DMA((2,)),
        ],
    )(logits, t)
